# baseline (speedup 1.0000x reference)
import functools

import jax
import jax.numpy as jnp
from jax import lax
from jax.experimental import pallas as pl
from jax.experimental.pallas import tpu as pltpu
from jax.experimental.pallas import tpu_sc as plsc

N_USERS = 1000000
N_STATEMENTS = 100000
N_FACTORS = 16
BATCH = 16384

NC = 2
NS = 16
NW = NC * NS
B_PER_W = BATCH // NW
CHUNK = 128
N_CHUNKS = B_PER_W // CHUNK
L = 16
N_GROUPS = B_PER_W // L


def _mf_kernel(uidx_hbm, sidx_hbm, uf_hbm, sf_hbm, ui_hbm, si_hbm, g_hbm,
               out_hbm,
               uidx_v, sidx_v, urows_v, srows_v, uint_v, sint_v, g_v, acc_v,
               sem):
    wid = lax.axis_index("s") * NC + lax.axis_index("c")
    base = wid * B_PER_W

    pltpu.sync_copy(uidx_hbm.at[wid], uidx_v)
    pltpu.sync_copy(sidx_hbm.at[wid], sidx_v)
    pltpu.sync_copy(g_hbm, g_v)

    copies = []
    for c in range(N_CHUNKS):
        sl = pl.ds(c * CHUNK, CHUNK)
        copies.append(pltpu.async_copy(uf_hbm.at[uidx_v.at[c]], urows_v.at[sl], sem))
        copies.append(pltpu.async_copy(sf_hbm.at[sidx_v.at[c]], srows_v.at[sl], sem))
        copies.append(pltpu.async_copy(ui_hbm.at[uidx_v.at[c]], uint_v.at[sl], sem))
        copies.append(pltpu.async_copy(si_hbm.at[sidx_v.at[c]], sint_v.at[sl], sem))
    for cp in copies:
        cp.wait()

    g_splat = g_v[...]
    lane = lax.iota(jnp.int32, L)

    def group_body(g, _):
        row_ids = g * L + lane
        acc = g_splat + uint_v[pl.ds(g * L, L)] + sint_v[pl.ds(g * L, L)]
        for j in range(N_FACTORS):
            colj = jnp.full((L,), j, jnp.int32)
            u = plsc.load_gather(urows_v, [row_ids, colj])
            s = plsc.load_gather(srows_v, [row_ids, colj])
            acc = acc + u * s
        acc_v[pl.ds(g * L, L)] = acc
        return _

    lax.fori_loop(0, N_GROUPS, group_body, 0)

    pltpu.sync_copy(acc_v, out_hbm.at[pl.ds(base, B_PER_W)])


@jax.jit
def _mf(uidx, sidx, uf, sf, ui, si, g):
    mesh = plsc.VectorSubcoreMesh(core_axis_name="c", subcore_axis_name="s")
    kern = functools.partial(
        pl.kernel,
        mesh=mesh,
        out_type=jax.ShapeDtypeStruct((BATCH,), jnp.float32),
        compiler_params=pltpu.CompilerParams(
            needs_layout_passes=False, use_tc_tiling_on_sc=False),
        scratch_types=[
            pltpu.VMEM((N_CHUNKS, CHUNK), jnp.int32),
            pltpu.VMEM((N_CHUNKS, CHUNK), jnp.int32),
            pltpu.VMEM((B_PER_W, N_FACTORS), jnp.float32),
            pltpu.VMEM((B_PER_W, N_FACTORS), jnp.float32),
            pltpu.VMEM((B_PER_W,), jnp.float32),
            pltpu.VMEM((B_PER_W,), jnp.float32),
            pltpu.VMEM((L,), jnp.float32),
            pltpu.VMEM((B_PER_W,), jnp.float32),
            pltpu.SemaphoreType.DMA,
        ],
    )(_mf_kernel)
    return kern(uidx, sidx, uf, sf, ui, si, g)


def kernel(user_indexes, statement_indexes, user_factors, statement_factors,
           user_intercepts, statement_intercepts, global_intercept):
    uidx = user_indexes.astype(jnp.int32).reshape(NW, N_CHUNKS, CHUNK)
    sidx = statement_indexes.astype(jnp.int32).reshape(NW, N_CHUNKS, CHUNK)
    ui = user_intercepts.reshape(N_USERS)
    si = statement_intercepts.reshape(N_STATEMENTS)
    g = jnp.broadcast_to(global_intercept.reshape(()), (L,))
    return _mf(uidx, sidx, user_factors, statement_factors, ui, si, g)


# speedup vs baseline: 2.0392x; 2.0392x over previous
import functools

import jax
import jax.numpy as jnp
from jax import lax
from jax.experimental import pallas as pl
from jax.experimental.pallas import tpu as pltpu
from jax.experimental.pallas import tpu_sc as plsc

N_USERS = 1000000
N_STATEMENTS = 100000
N_FACTORS = 16
BATCH = 16384

NC = 2
NS = 16
NW = NC * NS
B_PER_W = BATCH // NW
CHUNK = 128
N_CHUNKS = B_PER_W // CHUNK
L = 16
N_GROUPS = B_PER_W // L
UI_ROWS = (N_USERS + 127) // 128
SI_ROWS = (N_STATEMENTS + 127) // 128


def _mf_kernel(uidx_hbm, sidx_hbm, uf_hbm, sf_hbm, ui_hbm, si_hbm, g_hbm,
               out_hbm,
               uidx_v, sidx_v, utile_v, stile_v, uint_v,
               sint_v, g_v, acc_v, sem):
    wid = lax.axis_index("s") * NC + lax.axis_index("c")
    base = wid * B_PER_W

    pltpu.sync_copy(uidx_hbm.at[wid], uidx_v)
    pltpu.sync_copy(sidx_hbm.at[wid], sidx_v)
    pltpu.sync_copy(g_hbm, g_v)

    lane = lax.iota(jnp.int32, L)

    def idx_vec(ref, g):
        return ref[g // 8, pl.ds((g % 8) * L, L)]

    def issue_group(g, buf):
        iuv = idx_vec(uidx_v, g)
        isv = idx_vec(sidx_v, g)
        tu = iuv >> 3
        ts = isv >> 3
        for i in range(L):
            pltpu.async_copy(uf_hbm.at[tu[i]], utile_v.at[buf, i], sem)
            pltpu.async_copy(sf_hbm.at[ts[i]], stile_v.at[buf, i], sem)
        pltpu.async_copy(ui_hbm.at[iuv >> 7], uint_v.at[buf], sem)
        pltpu.async_copy(si_hbm.at[isv >> 7], sint_v.at[buf], sem)

    def wait_group(buf):
        for i in range(L):
            pltpu.make_async_copy(uf_hbm.at[0], utile_v.at[buf, i], sem).wait()
            pltpu.make_async_copy(sf_hbm.at[0], stile_v.at[buf, i], sem).wait()
        pltpu.make_async_copy(ui_hbm.at[lane], uint_v.at[buf], sem).wait()
        pltpu.make_async_copy(si_hbm.at[lane], sint_v.at[buf], sem).wait()

    def compute_group(g, buf):
        iuv = idx_vec(uidx_v, g)
        isv = idx_vec(sidx_v, g)
        su = iuv & 7
        ss = isv & 7
        ub = utile_v.at[buf]
        sb = stile_v.at[buf]
        acc = g_v[...]
        acc = acc + plsc.load_gather(uint_v.at[buf], [lane, iuv & 127])
        acc = acc + plsc.load_gather(sint_v.at[buf], [lane, isv & 127])
        for k in range(N_FACTORS):
            colk = jnp.full((L,), k, jnp.int32)
            u = plsc.load_gather(ub, [lane, su, colk])
            s = plsc.load_gather(sb, [lane, ss, colk])
            acc = acc + u * s
        acc_v[pl.ds(g * L, L)] = acc

    issue_group(0, 0)

    def body(g, carry):
        buf = lax.rem(g, 2)
        wait_group(buf)

        @pl.when(g + 1 < N_GROUPS)
        def _issue_next():
            issue_group(g + 1, 1 - buf)

        compute_group(g, buf)
        return carry

    lax.fori_loop(0, N_GROUPS, body, 0)

    pltpu.sync_copy(acc_v, out_hbm.at[pl.ds(base, B_PER_W)])


@jax.jit
def _mf(uidx, sidx, uf3, sf3, ui2, si2, g):
    mesh = plsc.VectorSubcoreMesh(core_axis_name="c", subcore_axis_name="s")
    kern = functools.partial(
        pl.kernel,
        mesh=mesh,
        out_type=jax.ShapeDtypeStruct((BATCH,), jnp.float32),
        compiler_params=pltpu.CompilerParams(needs_layout_passes=False),
        scratch_types=[
            pltpu.VMEM((N_CHUNKS, CHUNK), jnp.int32),
            pltpu.VMEM((N_CHUNKS, CHUNK), jnp.int32),
            pltpu.VMEM((2, L, 8, N_FACTORS), jnp.float32),
            pltpu.VMEM((2, L, 8, N_FACTORS), jnp.float32),
            pltpu.VMEM((2, L, 128), jnp.float32),
            pltpu.VMEM((2, L, 128), jnp.float32),
            pltpu.VMEM((L,), jnp.float32),
            pltpu.VMEM((B_PER_W,), jnp.float32),
            pltpu.SemaphoreType.DMA,
        ],
    )(_mf_kernel)
    return kern(uidx, sidx, uf3, sf3, ui2, si2, g)


def kernel(user_indexes, statement_indexes, user_factors, statement_factors,
           user_intercepts, statement_intercepts, global_intercept):
    uidx = user_indexes.astype(jnp.int32).reshape(NW, N_CHUNKS, CHUNK)
    sidx = statement_indexes.astype(jnp.int32).reshape(NW, N_CHUNKS, CHUNK)
    uf3 = user_factors.reshape(N_USERS // 8, 8, N_FACTORS)
    sf3 = statement_factors.reshape(N_STATEMENTS // 8, 8, N_FACTORS)
    ui2 = jnp.pad(user_intercepts.reshape(N_USERS),
                  (0, UI_ROWS * 128 - N_USERS)).reshape(UI_ROWS, 128)
    si2 = jnp.pad(statement_intercepts.reshape(N_STATEMENTS),
                  (0, SI_ROWS * 128 - N_STATEMENTS)).reshape(SI_ROWS, 128)
    g = jnp.broadcast_to(global_intercept.reshape(()), (L,))
    return _mf(uidx, sidx, uf3, sf3, ui2, si2, g)
